# Initial kernel scaffold; baseline (speedup 1.0000x reference)
import jax
import jax.numpy as jnp
from jax.experimental import pallas as pl


def kernel(params, x, edge_index, edge_attr, batch, node2motif, motif_edge_index, motif2graph):
    raise NotImplementedError("write your pallas kernel here")


# speedup vs baseline: 9.0248x; 9.0248x over previous
import functools

import jax
import jax.numpy as jnp
from jax import lax
from jax.experimental import pallas as pl
from jax.experimental.pallas import tpu as pltpu
from jax.experimental.pallas import tpu_sc as plsc

F32 = jnp.float32
I32 = jnp.int32

_N = 10000
_E = 320000
_B = 128
_M = 2000
_EM = 16000
_H = 128

_NP = 10240
_MP = 2048
_EMP = 16384
_RB = 256

_NC = 2
_NS = 16
_NW = _NC * _NS

_EPT = _E // _NW
_CE = 80
_NCH_E = _EPT // _CE

_MEPT = _EMP // _NS
_CM = 64
_NCH_M = _MEPT // _CM

_NPT = _NP // _NS
_CN = 64
_NCH_N = _NPT // _CN

_ASL = _NP // _NS
_MSL = _MP // _NS


def _sc_mesh():
    return plsc.VectorSubcoreMesh(core_axis_name="c", subcore_axis_name="s")


def _zero_rows(buf, nrows):
    def zrow(i, _):
        for j in range(_H // 16):
            buf[i, pl.ds(j * 16, 16)] = jnp.zeros((16,), F32)
        return 0
    lax.fori_loop(0, nrows, zrow, 0)


def _edge_call(h, t, src, codes, dst):
    @functools.partial(
        pl.kernel,
        out_type=jax.ShapeDtypeStruct((_NC * _NP, _H), F32),
        mesh=_sc_mesh(),
        scratch_types=[
            pltpu.VMEM((_CE,), I32),
            pltpu.VMEM((_CE,), I32),
            pltpu.VMEM((_CE,), I32),
            pltpu.VMEM((_CE, _H), F32),
            pltpu.VMEM((_CE, _H), F32),
            pltpu.VMEM_SHARED((_NP, _H), F32),
            pltpu.SemaphoreType.DMA,
            pltpu.SemaphoreType.DMA,
        ],
    )
    def k(h_hbm, t_hbm, src_hbm, code_hbm, dst_hbm, out_hbm,
          sidx, cidx, didx, hrows, trows, agg, sem1, sem2):
        cid = lax.axis_index("c")
        sid = lax.axis_index("s")
        wid = sid * _NC + cid
        _zero_rows(hrows, _CE)
        for r in range(_ASL // _CE):
            pltpu.sync_copy(hrows, agg.at[pl.ds(sid * _ASL + r * _CE, _CE), :])
        plsc.subcore_barrier()

        def chunk(kk, _):
            base = pl.multiple_of(wid * _EPT + kk * _CE, 8)
            pltpu.sync_copy(src_hbm.at[pl.ds(base, _CE)], sidx)
            pltpu.sync_copy(code_hbm.at[pl.ds(base, _CE)], cidx)
            pltpu.sync_copy(dst_hbm.at[pl.ds(base, _CE)], didx)
            pltpu.async_copy(h_hbm.at[sidx], hrows, sem1).wait()
            pltpu.async_copy(t_hbm.at[cidx], trows, sem2).wait()

            def vrow(i, _):
                for j in range(_H // 16):
                    s = pl.ds(j * 16, 16)
                    hrows[i, s] = jnp.maximum(hrows[i, s] + trows[i, s], 0.0)
                return 0
            lax.fori_loop(0, _CE, vrow, 0)
            pltpu.sync_copy(hrows, agg.at[didx], add=True)
            return 0
        lax.fori_loop(0, _NCH_E, chunk, 0)
        plsc.subcore_barrier()
        pltpu.sync_copy(
            agg.at[pl.ds(sid * _ASL, _ASL), :],
            out_hbm.at[pl.ds(cid * _NP + sid * _ASL, _ASL), :])

    return k(h, t, src, codes, dst)


def _node_scatter_call(h3, n2m):
    @functools.partial(
        pl.kernel,
        out_type=jax.ShapeDtypeStruct((_MP, _H), F32),
        mesh=_sc_mesh(),
        scratch_types=[
            pltpu.VMEM((_CN,), I32),
            pltpu.VMEM((_CN, _H), F32),
            pltpu.VMEM_SHARED((_MP, _H), F32),
        ],
    )
    def k(h_hbm, idx_hbm, out_hbm, idxv, rows, agg):
        cid = lax.axis_index("c")
        sid = lax.axis_index("s")

        @pl.when(cid == 0)
        def _():
            _zero_rows(rows, _CN)
            for r in range(_MSL // _CN):
                pltpu.sync_copy(rows, agg.at[pl.ds(sid * _MSL + r * _CN, _CN), :])
        plsc.subcore_barrier()

        @pl.when(cid == 0)
        def _():
            def chunk(kk, _):
                row0 = pl.multiple_of(sid * _NPT + kk * _CN, 8)
                pltpu.sync_copy(h_hbm.at[pl.ds(row0, _CN), :], rows)
                pltpu.sync_copy(idx_hbm.at[pl.ds(row0, _CN)], idxv)
                pltpu.sync_copy(rows, agg.at[idxv], add=True)
                return 0
            lax.fori_loop(0, _NCH_N, chunk, 0)
        plsc.subcore_barrier()

        @pl.when(cid == 0)
        def _():
            pltpu.sync_copy(agg.at[pl.ds(sid * _MSL, _MSL), :],
                            out_hbm.at[pl.ds(sid * _MSL, _MSL), :])

    return k(h3, n2m)


def _motif_edge_call(mh, ms, md):
    @functools.partial(
        pl.kernel,
        out_type=jax.ShapeDtypeStruct((_MP, _H), F32),
        mesh=_sc_mesh(),
        scratch_types=[
            pltpu.VMEM((_CM,), I32),
            pltpu.VMEM((_CM,), I32),
            pltpu.VMEM((_CM, _H), F32),
            pltpu.VMEM_SHARED((_MP, _H), F32),
            pltpu.SemaphoreType.DMA,
        ],
    )
    def k(mh_hbm, ms_hbm, md_hbm, out_hbm, idxs, idxd, rows, agg, sem):
        cid = lax.axis_index("c")
        sid = lax.axis_index("s")

        @pl.when(cid == 0)
        def _():
            _zero_rows(rows, _CM)
            for r in range(_MSL // _CM):
                pltpu.sync_copy(rows, agg.at[pl.ds(sid * _MSL + r * _CM, _CM), :])
        plsc.subcore_barrier()

        @pl.when(cid == 0)
        def _():
            def chunk(kk, _):
                base = pl.multiple_of(sid * _MEPT + kk * _CM, 8)
                pltpu.sync_copy(ms_hbm.at[pl.ds(base, _CM)], idxs)
                pltpu.sync_copy(md_hbm.at[pl.ds(base, _CM)], idxd)
                pltpu.async_copy(mh_hbm.at[idxs], rows, sem).wait()
                pltpu.sync_copy(rows, agg.at[idxd], add=True)
                return 0
            lax.fori_loop(0, _NCH_M, chunk, 0)
        plsc.subcore_barrier()

        @pl.when(cid == 0)
        def _():
            pltpu.sync_copy(agg.at[pl.ds(sid * _MSL, _MSL), :],
                            out_hbm.at[pl.ds(sid * _MSL, _MSL), :])

    return k(mh, ms, md)


def _atom_call(xoff, tab):
    grid = _NP // _RB

    def body(x_r, tab_r, out_r):
        cols = lax.broadcasted_iota(I32, (_RB, 576), 1)
        oh = jnp.zeros((_RB, 576), F32)
        for f in range(9):
            oh = oh + (x_r[:, f:f + 1] == cols).astype(F32)
        out_r[...] = jnp.dot(oh, tab_r[...], preferred_element_type=F32)

    return pl.pallas_call(
        body,
        grid=(grid,),
        in_specs=[pl.BlockSpec((_RB, 9), lambda i: (i, 0)),
                  pl.BlockSpec((576, _H), lambda i: (0, 0))],
        out_specs=pl.BlockSpec((_RB, _H), lambda i: (i, 0)),
        out_shape=jax.ShapeDtypeStruct((_NP, _H), F32),
    )(xoff, tab)


def _mlp_pool(h, aggs, w1, b1, w2, b2, eps, seg, rows_pad):
    grid = rows_pad // _RB
    na = len(aggs)
    epsf = jnp.full((8, _H), 1.0 + eps, F32)

    def body(*refs):
        eps_r = refs[0]
        h_r = refs[1]
        a_rs = refs[2:2 + na]
        w1_r, b1_r, w2_r, b2_r, seg_r = refs[2 + na:7 + na]
        ho_r, pool_r = refs[7 + na], refs[8 + na]
        i = pl.program_id(0)
        z = eps_r[0, 0] * h_r[...]
        for a_r in a_rs:
            z = z + a_r[...]
        u = jnp.maximum(jnp.dot(z, w1_r[...], preferred_element_type=F32) + b1_r[...], 0.0)
        v = jnp.maximum(jnp.dot(u, w2_r[...], preferred_element_type=F32) + b2_r[...], 0.0)
        ho_r[...] = v
        oh = (seg_r[...] == lax.broadcasted_iota(I32, (_RB, _B), 1)).astype(F32)
        p = lax.dot_general(oh, v, (((0,), (0,)), ((), ())),
                            preferred_element_type=F32)

        @pl.when(i == 0)
        def _():
            pool_r[...] = p

        @pl.when(i > 0)
        def _():
            pool_r[...] = pool_r[...] + p

    in_specs = [
        pl.BlockSpec((8, _H), lambda i: (0, 0)),
        pl.BlockSpec((_RB, _H), lambda i: (i, 0)),
        *[pl.BlockSpec((_RB, _H), lambda i: (i, 0)) for _ in aggs],
        pl.BlockSpec((_H, 2 * _H), lambda i: (0, 0)),
        pl.BlockSpec((1, 2 * _H), lambda i: (0, 0)),
        pl.BlockSpec((2 * _H, _H), lambda i: (0, 0)),
        pl.BlockSpec((1, _H), lambda i: (0, 0)),
        pl.BlockSpec((_RB, 1), lambda i: (i, 0)),
    ]
    out_specs = [
        pl.BlockSpec((_RB, _H), lambda i: (i, 0)),
        pl.BlockSpec((_B, _H), lambda i: (0, 0)),
    ]
    return pl.pallas_call(
        body,
        grid=(grid,),
        in_specs=in_specs,
        out_specs=out_specs,
        out_shape=[jax.ShapeDtypeStruct((rows_pad, _H), F32),
                   jax.ShapeDtypeStruct((_B, _H), F32)],
    )(epsf, h, *aggs, w1, b1.reshape(1, -1), w2, b2.reshape(1, -1), seg)


def kernel(params, x, edge_index, edge_attr, batch, node2motif,
           motif_edge_index, motif2graph):
    p = params
    atab = jnp.pad(p["atom"], ((0, 0), (0, 14), (0, 0))).reshape(9 * 64, _H)
    bond = p["bond"]
    ci = jnp.arange(512, dtype=I32)
    tcomb = bond[0][ci // 64] + bond[1][(ci // 8) % 8] + bond[2][ci % 8]
    x_p = jnp.pad(x.astype(I32), ((0, _NP - _N), (0, 0)))
    xoff = x_p + 64 * jnp.arange(9, dtype=I32)[None, :]
    src = edge_index[0].astype(I32)
    dst = edge_index[1].astype(I32)
    ea = edge_attr.astype(I32)
    codes = ea[:, 0] * 64 + ea[:, 1] * 8 + ea[:, 2]
    batch_p = jnp.pad(batch.astype(I32), (0, _NP - _N),
                      constant_values=_B).reshape(-1, 1)
    n2m_p = jnp.pad(node2motif.astype(I32), (0, _NP - _N),
                    constant_values=_MP - 1)
    ms_p = jnp.pad(motif_edge_index[0].astype(I32), (0, _EMP - _EM),
                   constant_values=_MP - 1)
    md_p = jnp.pad(motif_edge_index[1].astype(I32), (0, _EMP - _EM),
                   constant_values=_MP - 1)
    m2g_p = jnp.pad(motif2graph.astype(I32), (0, _MP - _M),
                    constant_values=_B).reshape(-1, 1)

    h = _atom_call(xoff, atab)
    gpools = []
    for lp in p["gc"]:
        agg2 = _edge_call(h, tcomb, src, codes, dst)
        h, gp = _mlp_pool(h, [agg2[:_NP], agg2[_NP:]], lp["W1"], lp["b1"],
                          lp["W2"], lp["b2"], lp["eps"], batch_p, _NP)
        gpools.append(gp)
    graph_embs = jnp.concatenate(gpools, axis=-1)

    mh = _node_scatter_call(h, n2m_p)
    mpools = []
    for lp in p["mc"]:
        agg = _motif_edge_call(mh, ms_p, md_p)
        mh, mp_ = _mlp_pool(mh, [agg], lp["W1"], lp["b1"], lp["W2"],
                            lp["b2"], lp["eps"], m2g_p, _MP)
        mpools.append(mp_)
    motif_embs = jnp.concatenate(mpools, axis=-1)
    return motif_embs, graph_embs
